# baseline (speedup 1.0000x reference)
import functools

import jax
import jax.numpy as jnp
import numpy as np
from jax import lax
from jax.experimental import pallas as pl
from jax.experimental.pallas import tpu as pltpu
from jax.experimental.pallas import tpu_sc as plsc

_VOCAB = 100000
_DIM = 128
_SCALE = float(np.sqrt(np.float32(_DIM)))

_B = 4096 * 50
_NC, _NS, _L = 2, 16, 16
_NW = _NC * _NS
_BPW = _B // _NW
_CHUNK = 400
_NCHUNK = _BPW // _CHUNK

_mesh = plsc.VectorSubcoreMesh(core_axis_name="c", subcore_axis_name="s")


@functools.partial(
    pl.kernel,
    mesh=_mesh,
    out_type=jax.ShapeDtypeStruct((_B, _DIM), jnp.float32),
    scratch_types=[
        pltpu.VMEM((_BPW,), jnp.int32),
        pltpu.VMEM((_CHUNK, _DIM), jnp.float32),
        pltpu.VMEM((_CHUNK, _DIM), jnp.float32),
        pltpu.SemaphoreType.DMA,
        pltpu.SemaphoreType.DMA,
        pltpu.SemaphoreType.DMA,
        pltpu.SemaphoreType.DMA,
    ],
)
def _emb_lookup(words_hbm, table_hbm, out_hbm, idx_v, buf0, buf1, g0, g1, s0, s1):
    wid = lax.axis_index("s") * _NC + lax.axis_index("c")
    base = wid * _BPW
    pltpu.sync_copy(words_hbm.at[pl.ds(base, _BPW)], idx_v)

    bufs = (buf0, buf1)
    gsems = (g0, g1)
    ssems = (s0, s1)

    def start_gather(c):
        off = c * _CHUNK
        return pltpu.async_copy(
            table_hbm.at[idx_v.at[pl.ds(off, _CHUNK)]], bufs[c % 2], gsems[c % 2]
        )

    def scale(buf):
        def scale_body(r, carry):
            for j in range(_DIM // _L):
                sl = pl.ds(j * _L, _L)
                buf[r, sl] = buf[r, sl] * _SCALE
            return carry

        lax.fori_loop(0, _CHUNK, scale_body, 0)

    gathers = [None] * _NCHUNK
    scatters = [None] * _NCHUNK
    gathers[0] = start_gather(0)
    for c in range(_NCHUNK):
        cb = c % 2
        if c + 1 < _NCHUNK:
            if c >= 1:
                scatters[c - 1].wait()
            gathers[c + 1] = start_gather(c + 1)
        gathers[c].wait()
        scale(bufs[cb])
        scatters[c] = pltpu.async_copy(
            bufs[cb], out_hbm.at[pl.ds(base + c * _CHUNK, _CHUNK)], ssems[cb]
        )
    scatters[_NCHUNK - 2].wait()
    scatters[_NCHUNK - 1].wait()


def kernel(words, table):
    flat = words.reshape(-1).astype(jnp.int32)
    out = _emb_lookup(flat, table)
    return out.reshape(words.shape + (_DIM,))


# speedup vs baseline: 1.7527x; 1.7527x over previous
import functools

import jax
import jax.numpy as jnp
import numpy as np
from jax import lax
from jax.experimental import pallas as pl
from jax.experimental.pallas import tpu as pltpu
from jax.experimental.pallas import tpu_sc as plsc

_VOCAB = 100000
_DIM = 128
_SEQ = 50
_NSEQ = 4096
_SCALE = float(np.sqrt(np.float32(_DIM)))

_B = _NSEQ * _SEQ
_NC, _NS, _L = 2, 16, 16
_NW = _NC * _NS
_BPW = _B // _NW
_SPW = _NSEQ // _NW
_CHUNK = 400
_CSEQ = _CHUNK // _SEQ
_NCHUNK = _BPW // _CHUNK

_mesh = plsc.VectorSubcoreMesh(core_axis_name="c", subcore_axis_name="s")


@functools.partial(
    pl.kernel,
    mesh=_mesh,
    out_type=jax.ShapeDtypeStruct((_NSEQ, _SEQ, _DIM), jnp.float32),
    scratch_types=[
        pltpu.VMEM((_BPW,), jnp.int32),
        pltpu.VMEM((_CHUNK, _DIM), jnp.float32),
        pltpu.VMEM((_CHUNK, _DIM), jnp.float32),
        pltpu.SemaphoreType.DMA,
        pltpu.SemaphoreType.DMA,
        pltpu.SemaphoreType.DMA,
        pltpu.SemaphoreType.DMA,
    ],
)
def _emb_lookup(words_hbm, table_hbm, out_hbm, idx_v, buf0, buf1, g0, g1, s0, s1):
    wid = lax.axis_index("s") * _NC + lax.axis_index("c")
    base = wid * _BPW
    seq_base = wid * _SPW
    pltpu.sync_copy(words_hbm.at[pl.ds(base, _BPW)], idx_v)

    bufs = (buf0, buf1)
    gsems = (g0, g1)
    ssems = (s0, s1)

    def start_gather(c):
        off = c * _CHUNK
        return pltpu.async_copy(
            table_hbm.at[idx_v.at[pl.ds(off, _CHUNK)]], bufs[c % 2], gsems[c % 2]
        )

    def scale(buf):
        def scale_body(r, carry):
            for j in range(_DIM // _L):
                sl = pl.ds(j * _L, _L)
                buf[r, sl] = buf[r, sl] * _SCALE
            return carry

        lax.fori_loop(0, _CHUNK, scale_body, 0)

    def start_scatters(c):
        cb = c % 2
        return [
            pltpu.async_copy(
                bufs[cb].at[pl.ds(i * _SEQ, _SEQ)],
                out_hbm.at[seq_base + c * _CSEQ + i],
                ssems[cb],
            )
            for i in range(_CSEQ)
        ]

    gathers = [None] * _NCHUNK
    scatters = [None] * _NCHUNK
    gathers[0] = start_gather(0)
    for c in range(_NCHUNK):
        cb = c % 2
        if c + 1 < _NCHUNK:
            if c >= 1:
                for cp in scatters[c - 1]:
                    cp.wait()
            gathers[c + 1] = start_gather(c + 1)
        gathers[c].wait()
        scale(bufs[cb])
        scatters[c] = start_scatters(c)
    for c in (_NCHUNK - 2, _NCHUNK - 1):
        for cp in scatters[c]:
            cp.wait()


def kernel(words, table):
    flat = words.reshape(-1).astype(jnp.int32)
    return _emb_lookup(flat, table)


# speedup vs baseline: 1.7549x; 1.0013x over previous
import functools

import jax
import jax.numpy as jnp
import numpy as np
from jax import lax
from jax.experimental import pallas as pl
from jax.experimental.pallas import tpu as pltpu
from jax.experimental.pallas import tpu_sc as plsc

_VOCAB = 100000
_DIM = 128
_SEQ = 50
_NSEQ = 4096
_SCALE = float(np.sqrt(np.float32(_DIM)))

_B = _NSEQ * _SEQ
_NC, _NS, _L = 2, 16, 16
_NW = _NC * _NS
_BPW = _B // _NW
_SPW = _NSEQ // _NW
_CHUNK = 400
_CSEQ = _CHUNK // _SEQ
_NCHUNK = _BPW // _CHUNK

_mesh = plsc.VectorSubcoreMesh(core_axis_name="c", subcore_axis_name="s")


@functools.partial(
    pl.kernel,
    mesh=_mesh,
    out_type=jax.ShapeDtypeStruct((_NSEQ, _SEQ, _DIM), jnp.float32),
    compiler_params=pltpu.CompilerParams(use_tc_tiling_on_sc=True),
    scratch_types=[
        pltpu.VMEM((_BPW,), jnp.int32),
        pltpu.VMEM((_CHUNK, _DIM), jnp.float32),
        pltpu.VMEM((_CHUNK, _DIM), jnp.float32),
        pltpu.SemaphoreType.DMA,
        pltpu.SemaphoreType.DMA,
        pltpu.SemaphoreType.DMA,
        pltpu.SemaphoreType.DMA,
    ],
)
def _emb_lookup(words_hbm, table_hbm, out_hbm, idx_v, buf0, buf1, g0, g1, s0, s1):
    wid = lax.axis_index("s") * _NC + lax.axis_index("c")
    base = wid * _BPW
    seq_base = wid * _SPW
    pltpu.sync_copy(words_hbm.at[pl.ds(base, _BPW)], idx_v)

    bufs = (buf0, buf1)
    gsems = (g0, g1)
    ssems = (s0, s1)

    def start_gather(c):
        off = c * _CHUNK
        return pltpu.async_copy(
            table_hbm.at[idx_v.at[pl.ds(off, _CHUNK)]], bufs[c % 2], gsems[c % 2]
        )

    def scale(buf):
        def scale_body(r, carry):
            for j in range(_DIM // _L):
                sl = pl.ds(j * _L, _L)
                buf[r, sl] = buf[r, sl] * _SCALE
            return carry

        lax.fori_loop(0, _CHUNK, scale_body, 0)

    def start_scatters(c):
        cb = c % 2
        return [
            pltpu.async_copy(
                bufs[cb].at[pl.ds(i * _SEQ, _SEQ)],
                out_hbm.at[seq_base + c * _CSEQ + i],
                ssems[cb],
            )
            for i in range(_CSEQ)
        ]

    gathers = [None] * _NCHUNK
    scatters = [None] * _NCHUNK
    gathers[0] = start_gather(0)
    for c in range(_NCHUNK):
        cb = c % 2
        if c + 1 < _NCHUNK:
            if c >= 1:
                for cp in scatters[c - 1]:
                    cp.wait()
            gathers[c + 1] = start_gather(c + 1)
        gathers[c].wait()
        scale(bufs[cb])
        scatters[c] = start_scatters(c)
    for c in (_NCHUNK - 2, _NCHUNK - 1):
        for cp in scatters[c]:
            cp.wait()


def kernel(words, table):
    flat = words.reshape(-1).astype(jnp.int32)
    return _emb_lookup(flat, table)


# speedup vs baseline: 2.8751x; 1.6383x over previous
import functools

import jax
import jax.numpy as jnp
import numpy as np
from jax import lax
from jax.experimental import pallas as pl
from jax.experimental.pallas import tpu as pltpu
from jax.experimental.pallas import tpu_sc as plsc

_VOCAB = 100000
_DIM = 128
_SEQ = 50
_NSEQ = 4096
_SCALE = float(np.sqrt(np.float32(_DIM)))

_NC, _NS, _L = 2, 16, 16
_NW = _NC * _NS
_BPW = _NSEQ // _NW

_mesh = plsc.VectorSubcoreMesh(core_axis_name="c", subcore_axis_name="s")


@functools.partial(
    pl.kernel,
    mesh=_mesh,
    out_type=jax.ShapeDtypeStruct((_SEQ, _NSEQ, _DIM), jnp.float32),
    scratch_types=[
        pltpu.VMEM((_SEQ, _BPW), jnp.int32),
        pltpu.VMEM((_BPW, _DIM), jnp.float32),
        pltpu.VMEM((_BPW, _DIM), jnp.float32),
        pltpu.SemaphoreType.DMA,
        pltpu.SemaphoreType.DMA,
        pltpu.SemaphoreType.DMA,
        pltpu.SemaphoreType.DMA,
    ],
)
def _emb_lookup(wordsT_hbm, table_hbm, out_hbm, idx_v, buf0, buf1, g0, g1, s0, s1):
    wid = lax.axis_index("s") * _NC + lax.axis_index("c")
    b0 = wid * _BPW
    pltpu.sync_copy(wordsT_hbm.at[:, pl.ds(b0, _BPW)], idx_v)

    bufs = (buf0, buf1)
    gsems = (g0, g1)
    ssems = (s0, s1)

    def start_gather(t):
        return pltpu.async_copy(
            table_hbm.at[idx_v.at[t]], bufs[t % 2], gsems[t % 2]
        )

    def scale(buf):
        def scale_body(r, carry):
            for j in range(_DIM // _L):
                sl = pl.ds(j * _L, _L)
                buf[r, sl] = buf[r, sl] * _SCALE
            return carry

        lax.fori_loop(0, _BPW, scale_body, 0)

    gathers = [None] * _SEQ
    writes = [None] * _SEQ
    gathers[0] = start_gather(0)
    for t in range(_SEQ):
        tb = t % 2
        if t + 1 < _SEQ:
            if t >= 1:
                writes[t - 1].wait()
            gathers[t + 1] = start_gather(t + 1)
        gathers[t].wait()
        scale(bufs[tb])
        writes[t] = pltpu.async_copy(
            bufs[tb], out_hbm.at[t, pl.ds(b0, _BPW)], ssems[tb]
        )
    writes[_SEQ - 2].wait()
    writes[_SEQ - 1].wait()


def kernel(words, table):
    wordsT = jnp.transpose(words).astype(jnp.int32)
    outT = _emb_lookup(wordsT, table)
    return jnp.transpose(outT, (1, 0, 2))


# speedup vs baseline: 3.1361x; 1.0908x over previous
import functools

import jax
import jax.numpy as jnp
import numpy as np
from jax import lax
from jax.experimental import pallas as pl
from jax.experimental.pallas import tpu as pltpu
from jax.experimental.pallas import tpu_sc as plsc

_VOCAB = 100000
_DIM = 128
_SEQ = 50
_NSEQ = 4096
_SCALE = float(np.sqrt(np.float32(_DIM)))

_NC, _NS, _L = 2, 16, 16
_NW = _NC * _NS
_BPW = _NSEQ // _NW

_mesh = plsc.VectorSubcoreMesh(core_axis_name="c", subcore_axis_name="s")


@functools.partial(
    pl.kernel,
    mesh=_mesh,
    out_type=jax.ShapeDtypeStruct((_SEQ, _NSEQ, _DIM), jnp.float32),
    scratch_types=[
        pltpu.VMEM((_SEQ, _BPW), jnp.int32),
        pltpu.VMEM((_BPW, _DIM), jnp.float32),
        pltpu.VMEM((_BPW, _DIM), jnp.float32),
        pltpu.VMEM((_BPW, _DIM), jnp.float32),
        pltpu.VMEM((_BPW, _DIM), jnp.float32),
        pltpu.SemaphoreType.DMA,
        pltpu.SemaphoreType.DMA,
        pltpu.SemaphoreType.DMA,
        pltpu.SemaphoreType.DMA,
        pltpu.SemaphoreType.DMA,
        pltpu.SemaphoreType.DMA,
        pltpu.SemaphoreType.DMA,
        pltpu.SemaphoreType.DMA,
    ],
)
def _emb_lookup(
    wordsT_hbm, table_hbm, out_hbm, idx_v,
    buf0, buf1, buf2, buf3, g0, g1, g2, g3, s0, s1, s2, s3,
):
    wid = lax.axis_index("s") * _NC + lax.axis_index("c")
    b0 = wid * _BPW
    pltpu.sync_copy(wordsT_hbm.at[:, pl.ds(b0, _BPW)], idx_v)

    bufs = (buf0, buf1, buf2, buf3)
    gsems = (g0, g1, g2, g3)
    ssems = (s0, s1, s2, s3)
    _NB = 4

    def start_gather(t):
        return pltpu.async_copy(
            table_hbm.at[idx_v.at[t]], bufs[t % _NB], gsems[t % _NB]
        )

    def scale(buf):
        def scale_body(r, carry):
            for j in range(_DIM // _L):
                sl = pl.ds(j * _L, _L)
                buf[r, sl] = buf[r, sl] * _SCALE
            return carry

        lax.fori_loop(0, _BPW, scale_body, 0)

    gathers = [None] * _SEQ
    writes = [None] * _SEQ
    for t in range(_NB - 1):
        gathers[t] = start_gather(t)
    for t in range(_SEQ):
        tb = t % _NB
        if t + _NB - 1 < _SEQ:
            if t >= 1:
                writes[t - 1].wait()
            gathers[t + _NB - 1] = start_gather(t + _NB - 1)
        gathers[t].wait()
        scale(bufs[tb])
        writes[t] = pltpu.async_copy(
            bufs[tb], out_hbm.at[t, pl.ds(b0, _BPW)], ssems[tb]
        )
    for t in range(_SEQ - _NB, _SEQ):
        writes[t].wait()


def kernel(words, table):
    wordsT = jnp.transpose(words).astype(jnp.int32)
    outT = _emb_lookup(wordsT, table)
    return jnp.transpose(outT, (1, 0, 2))
